# Initial kernel scaffold; baseline (speedup 1.0000x reference)
import jax
import jax.numpy as jnp
from jax.experimental import pallas as pl


def kernel(x, edge_index, batch, W0, b0, W1, b1, W2, b2, M0, mb0, M1, mb1):
    raise NotImplementedError("write your pallas kernel here")


# speedup vs baseline: 10.8145x; 10.8145x over previous
import functools

import jax
import jax.numpy as jnp
from jax import lax
from jax.experimental import pallas as pl
from jax.experimental.pallas import tpu as pltpu
from jax.experimental.pallas import tpu_sc as plsc

N = 10000
E = 320000
D = 128
G = 64

NC = 2
NS = 16
EC = E // NC
ET = EC // NS
K = 80
NCHUNK = ET // K

DEGW = 128

@functools.cache
def _mesh():
    return plsc.VectorSubcoreMesh(
        core_axis_name="c", subcore_axis_name="s",
        num_cores=NC, num_subcores=NS)


def _degree_body(dst_hbm, ones_hbm, zeros_hbm, out_hbm, dst_v, ones_v, acc):
    c = lax.axis_index("c")
    s = lax.axis_index("s")
    pltpu.sync_copy(ones_hbm, ones_v)

    @pl.when(s == 0)
    def _():
        pltpu.sync_copy(zeros_hbm, acc)

    plsc.subcore_barrier()
    ebase = c * EC + s * ET

    def chunk(i, carry):
        pltpu.sync_copy(dst_hbm.at[pl.ds(ebase + i * K, K)], dst_v)
        pltpu.sync_copy(ones_v, acc.at[dst_v], add=True)
        return carry

    lax.fori_loop(0, NCHUNK, chunk, 0)
    plsc.subcore_barrier()

    @pl.when(s == 0)
    def _():
        pltpu.sync_copy(acc, out_hbm.at[c])


@functools.cache
def _build_sc_degree():
    return pl.kernel(
        _degree_body,
        out_type=jax.ShapeDtypeStruct((NC, N, DEGW), jnp.float32),
        mesh=_mesh(),
        scratch_types=[
            pltpu.VMEM((K,), jnp.int32),
            pltpu.VMEM((K, DEGW), jnp.float32),
            pltpu.VMEM_SHARED((N, DEGW), jnp.float32),
        ],
    )


def _sc_degree(dst, ones_kw, zeros_nw):
    return _build_sc_degree()(dst, ones_kw, zeros_nw)


def _segsum_body(hs_hbm, src_hbm, dst_hbm, zeros_hbm, out_hbm,
                 src_v, dst_v, row_v, acc):
    c = lax.axis_index("c")
    s = lax.axis_index("s")

    @pl.when(s == 0)
    def _():
        pltpu.sync_copy(zeros_hbm, acc)

    plsc.subcore_barrier()
    ebase = c * EC + s * ET

    def chunk(i, carry):
        off = ebase + i * K
        pltpu.sync_copy(src_hbm.at[pl.ds(off, K)], src_v)
        pltpu.sync_copy(dst_hbm.at[pl.ds(off, K)], dst_v)
        pltpu.sync_copy(hs_hbm.at[src_v], row_v)
        pltpu.sync_copy(row_v, acc.at[dst_v], add=True)
        return carry

    lax.fori_loop(0, NCHUNK, chunk, 0)
    plsc.subcore_barrier()

    @pl.when(s == 0)
    def _():
        pltpu.sync_copy(acc, out_hbm.at[c])


@functools.cache
def _build_sc_segsum():
    return pl.kernel(
        _segsum_body,
        out_type=jax.ShapeDtypeStruct((NC, N, D), jnp.float32),
        mesh=_mesh(),
        scratch_types=[
            pltpu.VMEM((K,), jnp.int32),
            pltpu.VMEM((K,), jnp.int32),
            pltpu.VMEM((K, D), jnp.float32),
            pltpu.VMEM_SHARED((N, D), jnp.float32),
        ],
    )


def _sc_segsum(hs, src, dst, zeros_nd):
    return _build_sc_segsum()(hs, src, dst, zeros_nd)


def _dinv_body(deg_ref, out_ref):
    deg = deg_ref[0, :, 0:1] + deg_ref[1, :, 0:1] + 1.0
    out_ref[...] = lax.rsqrt(deg)


def _tc_dinv(deg_p):
    return pl.pallas_call(
        _dinv_body,
        out_shape=jax.ShapeDtypeStruct((N, 1), jnp.float32),
    )(deg_p)


_RB = 2000
_NB = N // _RB


def _mm0_body(x_ref, w_ref, dinv_ref, out_ref):
    h = jnp.dot(x_ref[...], w_ref[...], preferred_element_type=jnp.float32)
    out_ref[...] = h * dinv_ref[...]


def _tc_mm0(x, W0, dinv):
    return pl.pallas_call(
        _mm0_body,
        grid=(_NB,),
        in_specs=[
            pl.BlockSpec((_RB, D), lambda i: (i, 0)),
            pl.BlockSpec((D, D), lambda i: (0, 0)),
            pl.BlockSpec((_RB, 1), lambda i: (i, 0)),
        ],
        out_specs=pl.BlockSpec((_RB, D), lambda i: (i, 0)),
        out_shape=jax.ShapeDtypeStruct((N, D), jnp.float32),
    )(x, W0, dinv)


def _mm_body(p_ref, hs_ref, dinv_ref, b_ref, w_ref, out_ref):
    dinv = dinv_ref[...]
    x = dinv * (p_ref[0] + p_ref[1] + hs_ref[...]) + b_ref[...]
    x = jnp.maximum(x, 0.0)
    h = jnp.dot(x, w_ref[...], preferred_element_type=jnp.float32)
    out_ref[...] = h * dinv


def _tc_mm(p, hs, dinv, b, W):
    return pl.pallas_call(
        _mm_body,
        grid=(_NB,),
        in_specs=[
            pl.BlockSpec((NC, _RB, D), lambda i: (0, i, 0)),
            pl.BlockSpec((_RB, D), lambda i: (i, 0)),
            pl.BlockSpec((_RB, 1), lambda i: (i, 0)),
            pl.BlockSpec((1, D), lambda i: (0, 0)),
            pl.BlockSpec((D, D), lambda i: (0, 0)),
        ],
        out_specs=pl.BlockSpec((_RB, D), lambda i: (i, 0)),
        out_shape=jax.ShapeDtypeStruct((N, D), jnp.float32),
    )(p, hs, dinv, b, W)


def _final_body(p_ref, hs_ref, dinv_ref, b_ref, m0_ref, mb0_ref, m1_ref,
                mb1_ref, batch_ref, out_ref, acc_s, acc_c):
    i = pl.program_id(0)

    @pl.when(i == 0)
    def _():
        acc_s[...] = jnp.zeros_like(acc_s)
        acc_c[...] = jnp.zeros_like(acc_c)

    x = dinv_ref[...] * (p_ref[0] + p_ref[1] + hs_ref[...]) + b_ref[...]
    r = jnp.maximum(
        jnp.dot(x, m0_ref[...], preferred_element_type=jnp.float32)
        + mb0_ref[...], 0.0)
    y = jnp.dot(r, m1_ref[...], preferred_element_type=jnp.float32) \
        + mb1_ref[...]
    gids = lax.broadcasted_iota(jnp.int32, (1, G), 1)
    onehot = (batch_ref[...] == gids).astype(jnp.float32)
    acc_s[...] += lax.dot_general(
        onehot, y, (((0,), (0,)), ((), ())),
        preferred_element_type=jnp.float32)
    acc_c[...] += lax.dot_general(
        onehot, jnp.ones_like(y), (((0,), (0,)), ((), ())),
        preferred_element_type=jnp.float32)

    @pl.when(i == _NB - 1)
    def _():
        out_ref[...] = acc_s[...] / jnp.maximum(acc_c[...], 1.0)


def _tc_final(p, hs, dinv, b2, M0, mb0, M1, mb1, batch2d):
    return pl.pallas_call(
        _final_body,
        grid=(_NB,),
        in_specs=[
            pl.BlockSpec((NC, _RB, D), lambda i: (0, i, 0)),
            pl.BlockSpec((_RB, D), lambda i: (i, 0)),
            pl.BlockSpec((_RB, 1), lambda i: (i, 0)),
            pl.BlockSpec((1, D), lambda i: (0, 0)),
            pl.BlockSpec((D, D // 2), lambda i: (0, 0)),
            pl.BlockSpec((1, D // 2), lambda i: (0, 0)),
            pl.BlockSpec((D // 2, 1), lambda i: (0, 0)),
            pl.BlockSpec((1, 1), lambda i: (0, 0)),
            pl.BlockSpec((_RB, 1), lambda i: (i, 0)),
        ],
        out_specs=pl.BlockSpec((G, 1), lambda i: (0, 0)),
        out_shape=jax.ShapeDtypeStruct((G, 1), jnp.float32),
        scratch_shapes=[
            pltpu.VMEM((G, 1), jnp.float32),
            pltpu.VMEM((G, 1), jnp.float32),
        ],
    )(p, hs, dinv, b2, M0, mb0, M1, mb1, batch2d)


def kernel(x, edge_index, batch, W0, b0, W1, b1, W2, b2, M0, mb0, M1, mb1):
    src = edge_index[0]
    dst = edge_index[1]
    zeros_nd = jnp.zeros((N, D), jnp.float32)
    zeros_nw = jnp.zeros((N, DEGW), jnp.float32)
    ones_kw = jnp.ones((K, DEGW), jnp.float32)

    deg_p = _sc_degree(dst, ones_kw, zeros_nw)
    dinv = _tc_dinv(deg_p)

    hs = _tc_mm0(x, W0, dinv)
    p = _sc_segsum(hs, src, dst, zeros_nd)
    hs = _tc_mm(p, hs, dinv, b0.reshape(1, D), W1)
    p = _sc_segsum(hs, src, dst, zeros_nd)
    hs = _tc_mm(p, hs, dinv, b1.reshape(1, D), W2)
    p = _sc_segsum(hs, src, dst, zeros_nd)

    return _tc_final(p, hs, dinv, b2.reshape(1, D), M0, mb0.reshape(1, D // 2),
                     M1, mb1.reshape(1, 1), batch.reshape(N, 1))
